# baseline (speedup 1.0000x reference)
import functools

import jax
import jax.numpy as jnp
from jax import lax
from jax.experimental import pallas as pl
from jax.experimental.pallas import tpu as pltpu
from jax.experimental.pallas import tpu_sc as plsc

N = 10000
E = 320000
D = 128
EPS = 1e-5

NC = 2
NS = 16
NW = NC * NS
CH = 128
NCH = E // CH
PW0 = NCH // NW
PXT = NCH - PW0 * NW
P = PW0 + 1
ACCN = N
RPT = N // NS
DEGW = 8

_mesh = plsc.VectorSubcoreMesh(core_axis_name="c", subcore_axis_name="s")
_sc_params = pltpu.CompilerParams(use_tc_tiling_on_sc=False)


@functools.partial(
    pl.kernel,
    out_type=jax.ShapeDtypeStruct((NC, N, DEGW), jnp.float32),
    mesh=_mesh,
    scratch_types=[
        [pltpu.VMEM((CH,), jnp.int32)] * 4,
        pltpu.VMEM((CH, DEGW), jnp.float32),
        pltpu.VMEM_SHARED((ACCN, DEGW), jnp.float32),
        [pltpu.SemaphoreType.DMA] * 4,
        [pltpu.SemaphoreType.DMA] * 2,
    ],
    compiler_params=_sc_params,
)
def _deg_kernel(dst_hbm, ones_hbm, out_hbm, dst_v, ones_v, acc, isems, ssems):
    c = lax.axis_index("c")
    s = lax.axis_index("s")
    wid = s * NC + c
    pw = jnp.where(wid < PXT, PW0 + 1, PW0)
    cb = wid * PW0 + jnp.minimum(wid, PXT)
    pltpu.sync_copy(ones_hbm.at[pl.ds(s * RPT, RPT)], acc.at[pl.ds(s * RPT, RPT)])
    pltpu.sync_copy(ones_hbm.at[pl.ds(0, CH)], ones_v)
    plsc.subcore_barrier()

    def idx(ci, q):
        return pltpu.make_async_copy(dst_hbm.at[cb + ci], dst_v[q], isems[q])

    def scat(q, b):
        return pltpu.make_async_copy(ones_v, acc.at[dst_v[q]], ssems[b])

    idx(0, 0).start()
    idx(1, 1).start()

    def body(g, carry):
        for k4 in range(4):
            ci = 4 * g + k4
            k = k4 % 2

            @pl.when(ci < pw)
            def _():
                idx(ci, k4).wait()
                scat(k4, k).start(add=True)

            @pl.when(ci + 2 < pw)
            def _():
                idx(ci + 2, (k4 + 2) % 4).start()

            @pl.when((ci > 0) & (ci <= pw))
            def _():
                scat((k4 + 3) % 4, 1 - k).wait()

        return carry

    lax.fori_loop(0, (P + 4) // 4, body, 0)
    plsc.subcore_barrier()
    pltpu.sync_copy(acc.at[pl.ds(s * RPT, RPT)], out_hbm.at[c, pl.ds(s * RPT, RPT)])


@functools.partial(
    pl.kernel,
    out_type=jax.ShapeDtypeStruct((NC, N, D), jnp.float32),
    mesh=_mesh,
    scratch_types=[
        [pltpu.VMEM((CH,), jnp.int32)] * 6,
        [pltpu.VMEM((CH,), jnp.int32)] * 6,
        [pltpu.VMEM((CH, D), jnp.float32)] * 3,
        pltpu.VMEM_SHARED((ACCN, D), jnp.float32),
        [pltpu.SemaphoreType.DMA] * 6,
        [pltpu.SemaphoreType.DMA] * 3,
        [pltpu.SemaphoreType.DMA] * 2,
    ],
    compiler_params=_sc_params,
)
def _conv_kernel(xs_hbm, src_hbm, dst_hbm, out_hbm, src_v, dst_v, rows,
                 acc, isems, gsems, ssems):
    c = lax.axis_index("c")
    s = lax.axis_index("s")
    wid = s * NC + c
    pw = jnp.where(wid < PXT, PW0 + 1, PW0)
    cb = wid * PW0 + jnp.minimum(wid, PXT)

    def idx(ci, q):
        return (pltpu.make_async_copy(src_hbm.at[cb + ci], src_v[q], isems[q]),
                pltpu.make_async_copy(dst_hbm.at[cb + ci], dst_v[q], isems[q]))

    def gath(ci8, b4):
        return pltpu.make_async_copy(xs_hbm.at[src_v[ci8]], rows[b4], gsems[b4])

    def scat(ci8, b4, k):
        return pltpu.make_async_copy(rows[b4], acc.at[dst_v[ci8]], ssems[k])

    for q in range(4):
        for d in idx(q, q):
            d.start()
    pltpu.sync_copy(xs_hbm.at[pl.ds(s * RPT, RPT)], acc.at[pl.ds(s * RPT, RPT)])
    for d in idx(0, 0):
        d.wait()
    gath(0, 0).start()
    for d in idx(1, 1):
        d.wait()
    gath(1, 1).start()
    plsc.subcore_barrier()

    def body(g, carry):
        for k6 in range(6):
            ci = 6 * g + k6
            k3 = k6 % 3
            k = k6 % 2

            @pl.when(ci < pw)
            def _():
                gath(k6, k3).wait()

            @pl.when((ci >= 1) & (ci < pw + 1))
            def _():
                scat((k6 + 5) % 6, (k3 + 2) % 3, 1 - k).wait()

            @pl.when(ci < pw)
            def _():
                scat(k6, k3, k).start(add=True)

            @pl.when(ci + 2 < pw)
            def _():
                for d in idx(ci + 2, (k6 + 2) % 6):
                    d.wait()
                gath((k6 + 2) % 6, (k3 + 2) % 3).start()

            @pl.when(ci + 4 < pw)
            def _():
                for d in idx(ci + 4, (k6 + 4) % 6):
                    d.start()

        return carry

    lax.fori_loop(0, (P + 1 + 5) // 6, body, 0)
    plsc.subcore_barrier()
    pltpu.sync_copy(acc.at[pl.ds(s * RPT, RPT)], out_hbm.at[c, pl.ds(s * RPT, RPT)])


MB = 1000
NBLK = N // MB


def _prep_body(d0_ref, d1_ref, x_ref, dinv_ref, xs_ref):
    deg = d0_ref[...] + d1_ref[...] - 1.0
    dinv = lax.rsqrt(deg)
    dinv_ref[...] = dinv
    xs_ref[...] = x_ref[...] * dinv


_prep = pl.pallas_call(
    _prep_body,
    grid=(NBLK,),
    in_specs=[
        pl.BlockSpec((MB, 1), lambda i: (i, 0)),
        pl.BlockSpec((MB, 1), lambda i: (i, 0)),
        pl.BlockSpec((MB, D), lambda i: (i, 0)),
    ],
    out_specs=(
        pl.BlockSpec((MB, 1), lambda i: (i, 0)),
        pl.BlockSpec((MB, D), lambda i: (i, 0)),
    ),
    out_shape=(
        jax.ShapeDtypeStruct((N, 1), jnp.float32),
        jax.ShapeDtypeStruct((N, D), jnp.float32),
    ),
)


def _mmln_body(residual, scale_out, *refs):
    if residual:
        (p0_ref, p1_ref, xs_ref, dinv_ref, w_ref, b_ref, xres_ref,
         lnw_ref, lnb_ref, out_ref, h_scr, acc_ref) = refs
    else:
        (p0_ref, p1_ref, xs_ref, dinv_ref, w_ref, b_ref,
         lnw_ref, lnb_ref, out_ref, h_scr, acc_ref) = refs
    i = pl.program_id(0)

    @pl.when(i == 0)
    def _():
        acc_ref[0] = 0.0
        acc_ref[1] = 0.0

    @pl.when(i < NBLK)
    def _():
        t = p0_ref[0] + p1_ref[0] - xs_ref[...]
        z = t * dinv_ref[...]
        h = jnp.dot(z, w_ref[...], preferred_element_type=jnp.float32) + b_ref[...]
        if residual:
            h = h + xres_ref[...]
        h_scr[pl.ds(i * MB, MB), :] = h
        acc_ref[0] += jnp.sum(h)
        acc_ref[1] += jnp.sum(h * h)

    @pl.when(i >= NBLK)
    def _():
        inv_n = 1.0 / (N * D)
        mean = acc_ref[0] * inv_n
        var = acc_ref[1] * inv_n - mean * mean
        rstd = lax.rsqrt(var + EPS)
        h = h_scr[pl.ds((i - NBLK) * MB, MB), :]
        y = (h - mean) * rstd * lnw_ref[...] + lnb_ref[...]
        y = jnp.maximum(y, 0.0)
        if scale_out:
            y = y * dinv_ref[...]
        out_ref[...] = y


def _make_mmln(residual, scale_out):
    def ph1_map(i):
        return (jnp.minimum(i, NBLK - 1), 0)

    p0_spec = pl.BlockSpec((1, MB, D), lambda i: (0, jnp.minimum(i, NBLK - 1), 0))
    p1_spec = pl.BlockSpec((1, MB, D), lambda i: (1, jnp.minimum(i, NBLK - 1), 0))
    row1_spec = pl.BlockSpec((MB, D), ph1_map)
    dinv_spec = pl.BlockSpec((MB, 1), lambda i: (i % NBLK, 0))
    full_spec = pl.BlockSpec((D, D), lambda i: (0, 0))
    b_spec = pl.BlockSpec((1, D), lambda i: (0, 0))
    in_specs = [p0_spec, p1_spec, row1_spec, dinv_spec, full_spec, b_spec]
    if residual:
        in_specs.append(row1_spec)
    in_specs += [b_spec, b_spec]
    return pl.pallas_call(
        functools.partial(_mmln_body, residual, scale_out),
        grid=(2 * NBLK,),
        in_specs=in_specs,
        out_specs=pl.BlockSpec(
            (MB, D), lambda i: (jnp.where(i < NBLK, 0, i - NBLK), 0)),
        out_shape=jax.ShapeDtypeStruct((N, D), jnp.float32),
        scratch_shapes=[
            pltpu.VMEM((N, D), jnp.float32),
            pltpu.SMEM((2,), jnp.float32),
        ],
    )


_mmln0 = _make_mmln(False, True)
_mmln1 = _make_mmln(True, False)


def kernel(x, edge_index, W0, b0, W1, b1, ln0_w, ln0_b, ln1_w, ln1_b):
    src = edge_index[0].reshape(NCH, CH)
    dst = edge_index[1].reshape(NCH, CH)
    ones = jnp.ones((N, DEGW), jnp.float32)
    b0r = b0.reshape(1, D)
    b1r = b1.reshape(1, D)
    ln0w = ln0_w.reshape(1, D)
    ln0b = ln0_b.reshape(1, D)
    ln1w = ln1_w.reshape(1, D)
    ln1b = ln1_b.reshape(1, D)

    degp = _deg_kernel(dst, ones)
    dinv, xs0 = _prep(degp[0, :, 0:1], degp[1, :, 0:1], x)

    p = _conv_kernel(xs0, src, dst)
    xs1 = _mmln0(p, p, xs0, dinv, W0, b0r, ln0w, ln0b)

    q = _conv_kernel(xs1, src, dst)
    out = _mmln1(q, q, xs1, dinv, W1, b1r, x, ln1w, ln1b)
    return out


# speedup vs baseline: 1.0433x; 1.0433x over previous
import functools

import jax
import jax.numpy as jnp
from jax import lax
from jax.experimental import pallas as pl
from jax.experimental.pallas import tpu as pltpu
from jax.experimental.pallas import tpu_sc as plsc

N = 10000
E = 320000
D = 128
EPS = 1e-5

NC = 2
NS = 16
NW = NC * NS
CH = 128
NCH = E // CH
PW0 = NCH // NW
PXT = NCH - PW0 * NW
P = PW0 + 1
ACCN = N
RPT = N // NS
DEGW = 8

_mesh = plsc.VectorSubcoreMesh(core_axis_name="c", subcore_axis_name="s")
_sc_params = pltpu.CompilerParams(use_tc_tiling_on_sc=False)


@functools.partial(
    pl.kernel,
    out_type=jax.ShapeDtypeStruct((NC, N, DEGW), jnp.float32),
    mesh=_mesh,
    scratch_types=[
        [pltpu.VMEM((CH,), jnp.int32)] * 6,
        pltpu.VMEM((CH, DEGW), jnp.float32),
        pltpu.VMEM_SHARED((ACCN, DEGW), jnp.float32),
        [pltpu.SemaphoreType.DMA] * 6,
        [pltpu.SemaphoreType.DMA] * 4,
    ],
    compiler_params=_sc_params,
)
def _deg_kernel(dst_hbm, ones_hbm, out_hbm, dst_v, ones_v, acc, isems, ssems):
    c = lax.axis_index("c")
    s = lax.axis_index("s")
    wid = s * NC + c
    pw = jnp.where(wid < PXT, PW0 + 1, PW0)
    cb = wid * PW0 + jnp.minimum(wid, PXT)
    pltpu.sync_copy(ones_hbm.at[pl.ds(s * RPT, RPT)], acc.at[pl.ds(s * RPT, RPT)])
    pltpu.sync_copy(ones_hbm.at[pl.ds(0, CH)], ones_v)
    plsc.subcore_barrier()

    def idx(ci, q):
        return pltpu.make_async_copy(dst_hbm.at[cb + ci], dst_v[q], isems[q])

    def scat(q, b):
        return pltpu.make_async_copy(ones_v, acc.at[dst_v[q]], ssems[b])

    idx(0, 0).start()
    idx(1, 1).start()

    def body(g, carry):
        for k12 in range(12):
            ci = 12 * g + k12
            q6 = k12 % 6
            k4 = k12 % 4

            @pl.when(ci < pw)
            def _():
                idx(ci, q6).wait()

            @pl.when((ci >= 3) & (ci < pw + 3))
            def _():
                scat((q6 + 3) % 6, (k4 + 1) % 4).wait()

            @pl.when(ci < pw)
            def _():
                scat(q6, k4).start(add=True)

            @pl.when(ci + 2 < pw)
            def _():
                idx(ci + 2, (q6 + 2) % 6).start()

        return carry

    lax.fori_loop(0, (P + 3 + 11) // 12, body, 0)
    plsc.subcore_barrier()
    pltpu.sync_copy(acc.at[pl.ds(s * RPT, RPT)], out_hbm.at[c, pl.ds(s * RPT, RPT)])


@functools.partial(
    pl.kernel,
    out_type=jax.ShapeDtypeStruct((NC, N, D), jnp.float32),
    mesh=_mesh,
    scratch_types=[
        [pltpu.VMEM((CH,), jnp.int32)] * 6,
        [pltpu.VMEM((CH,), jnp.int32)] * 6,
        [pltpu.VMEM((CH, D), jnp.float32)] * 3,
        pltpu.VMEM_SHARED((ACCN, D), jnp.float32),
        [pltpu.SemaphoreType.DMA] * 6,
        [pltpu.SemaphoreType.DMA] * 3,
        [pltpu.SemaphoreType.DMA] * 2,
    ],
    compiler_params=_sc_params,
)
def _conv_kernel(xs_hbm, src_hbm, dst_hbm, out_hbm, src_v, dst_v, rows,
                 acc, isems, gsems, ssems):
    c = lax.axis_index("c")
    s = lax.axis_index("s")
    wid = s * NC + c
    pw = jnp.where(wid < PXT, PW0 + 1, PW0)
    cb = wid * PW0 + jnp.minimum(wid, PXT)

    def idx(ci, q):
        return (pltpu.make_async_copy(src_hbm.at[cb + ci], src_v[q], isems[q]),
                pltpu.make_async_copy(dst_hbm.at[cb + ci], dst_v[q], isems[q]))

    def gath(ci8, b4):
        return pltpu.make_async_copy(xs_hbm.at[src_v[ci8]], rows[b4], gsems[b4])

    def scat(ci8, b4, k):
        return pltpu.make_async_copy(rows[b4], acc.at[dst_v[ci8]], ssems[k])

    for q in range(4):
        for d in idx(q, q):
            d.start()
    pltpu.sync_copy(xs_hbm.at[pl.ds(s * RPT, RPT)], acc.at[pl.ds(s * RPT, RPT)])
    for d in idx(0, 0):
        d.wait()
    gath(0, 0).start()
    for d in idx(1, 1):
        d.wait()
    gath(1, 1).start()
    plsc.subcore_barrier()

    def body(g, carry):
        for k6 in range(6):
            ci = 6 * g + k6
            k3 = k6 % 3
            k = k6 % 2

            @pl.when(ci < pw)
            def _():
                gath(k6, k3).wait()

            @pl.when((ci >= 1) & (ci < pw + 1))
            def _():
                scat((k6 + 5) % 6, (k3 + 2) % 3, 1 - k).wait()

            @pl.when(ci < pw)
            def _():
                scat(k6, k3, k).start(add=True)

            @pl.when(ci + 2 < pw)
            def _():
                for d in idx(ci + 2, (k6 + 2) % 6):
                    d.wait()
                gath((k6 + 2) % 6, (k3 + 2) % 3).start()

            @pl.when(ci + 4 < pw)
            def _():
                for d in idx(ci + 4, (k6 + 4) % 6):
                    d.start()

        return carry

    lax.fori_loop(0, (P + 1 + 5) // 6, body, 0)
    plsc.subcore_barrier()
    pltpu.sync_copy(acc.at[pl.ds(s * RPT, RPT)], out_hbm.at[c, pl.ds(s * RPT, RPT)])


MB = 2000
NBLK = N // MB


def _prep_body(d0_ref, d1_ref, x_ref, dinv_ref, xs_ref):
    deg = d0_ref[...] + d1_ref[...] - 1.0
    dinv = lax.rsqrt(deg)
    dinv_ref[...] = dinv
    xs_ref[...] = x_ref[...] * dinv


_prep = pl.pallas_call(
    _prep_body,
    grid=(NBLK,),
    in_specs=[
        pl.BlockSpec((MB, 1), lambda i: (i, 0)),
        pl.BlockSpec((MB, 1), lambda i: (i, 0)),
        pl.BlockSpec((MB, D), lambda i: (i, 0)),
    ],
    out_specs=(
        pl.BlockSpec((MB, 1), lambda i: (i, 0)),
        pl.BlockSpec((MB, D), lambda i: (i, 0)),
    ),
    out_shape=(
        jax.ShapeDtypeStruct((N, 1), jnp.float32),
        jax.ShapeDtypeStruct((N, D), jnp.float32),
    ),
)


def _mmln_body(residual, scale_out, *refs):
    if residual:
        (p0_ref, p1_ref, xs_ref, dinv_ref, w_ref, b_ref, xres_ref,
         lnw_ref, lnb_ref, out_ref, h_scr, acc_ref) = refs
    else:
        (p0_ref, p1_ref, xs_ref, dinv_ref, w_ref, b_ref,
         lnw_ref, lnb_ref, out_ref, h_scr, acc_ref) = refs
    i = pl.program_id(0)

    @pl.when(i == 0)
    def _():
        acc_ref[0] = 0.0
        acc_ref[1] = 0.0

    @pl.when(i < NBLK)
    def _():
        t = p0_ref[0] + p1_ref[0] - xs_ref[...]
        z = t * dinv_ref[...]
        h = jnp.dot(z, w_ref[...], preferred_element_type=jnp.float32) + b_ref[...]
        if residual:
            h = h + xres_ref[...]
        h_scr[pl.ds(i * MB, MB), :] = h
        acc_ref[0] += jnp.sum(h)
        acc_ref[1] += jnp.sum(h * h)

    @pl.when(i >= NBLK)
    def _():
        inv_n = 1.0 / (N * D)
        mean = acc_ref[0] * inv_n
        var = acc_ref[1] * inv_n - mean * mean
        rstd = lax.rsqrt(var + EPS)
        h = h_scr[pl.ds((i - NBLK) * MB, MB), :]
        y = (h - mean) * rstd * lnw_ref[...] + lnb_ref[...]
        y = jnp.maximum(y, 0.0)
        if scale_out:
            y = y * dinv_ref[...]
        out_ref[...] = y


def _make_mmln(residual, scale_out):
    def ph1_map(i):
        return (jnp.minimum(i, NBLK - 1), 0)

    p0_spec = pl.BlockSpec((1, MB, D), lambda i: (0, jnp.minimum(i, NBLK - 1), 0))
    p1_spec = pl.BlockSpec((1, MB, D), lambda i: (1, jnp.minimum(i, NBLK - 1), 0))
    row1_spec = pl.BlockSpec((MB, D), ph1_map)
    dinv_spec = pl.BlockSpec((MB, 1), lambda i: (i % NBLK, 0))
    full_spec = pl.BlockSpec((D, D), lambda i: (0, 0))
    b_spec = pl.BlockSpec((1, D), lambda i: (0, 0))
    in_specs = [p0_spec, p1_spec, row1_spec, dinv_spec, full_spec, b_spec]
    if residual:
        in_specs.append(row1_spec)
    in_specs += [b_spec, b_spec]
    return pl.pallas_call(
        functools.partial(_mmln_body, residual, scale_out),
        grid=(2 * NBLK,),
        in_specs=in_specs,
        out_specs=pl.BlockSpec(
            (MB, D), lambda i: (jnp.where(i < NBLK, 0, i - NBLK), 0)),
        out_shape=jax.ShapeDtypeStruct((N, D), jnp.float32),
        scratch_shapes=[
            pltpu.VMEM((N, D), jnp.float32),
            pltpu.SMEM((2,), jnp.float32),
        ],
    )


_mmln0 = _make_mmln(False, True)
_mmln1 = _make_mmln(True, False)


def kernel(x, edge_index, W0, b0, W1, b1, ln0_w, ln0_b, ln1_w, ln1_b):
    src = edge_index[0].reshape(NCH, CH)
    dst = edge_index[1].reshape(NCH, CH)
    ones = jnp.ones((N, DEGW), jnp.float32)
    b0r = b0.reshape(1, D)
    b1r = b1.reshape(1, D)
    ln0w = ln0_w.reshape(1, D)
    ln0b = ln0_b.reshape(1, D)
    ln1w = ln1_w.reshape(1, D)
    ln1b = ln1_b.reshape(1, D)

    degp = _deg_kernel(dst, ones)
    dinv, xs0 = _prep(degp[0, :, 0:1], degp[1, :, 0:1], x)

    p = _conv_kernel(xs0, src, dst)
    xs1 = _mmln0(p, p, xs0, dinv, W0, b0r, ln0w, ln0b)

    q = _conv_kernel(xs1, src, dst)
    out = _mmln1(q, q, xs1, dinv, W1, b1r, x, ln1w, ln1b)
    return out


# speedup vs baseline: 1.0625x; 1.0184x over previous
import functools

import jax
import jax.numpy as jnp
from jax import lax
from jax.experimental import pallas as pl
from jax.experimental.pallas import tpu as pltpu
from jax.experimental.pallas import tpu_sc as plsc

N = 10000
E = 320000
D = 128
EPS = 1e-5

NC = 2
NS = 16
NW = NC * NS
CH = 128
NCH = E // CH
PW0 = NCH // NW
PXT = NCH - PW0 * NW
P = PW0 + 1
ACCN = N
RPT = N // NS
DEGW = 8

_mesh = plsc.VectorSubcoreMesh(core_axis_name="c", subcore_axis_name="s")
_sc_params = pltpu.CompilerParams(use_tc_tiling_on_sc=False)


@functools.partial(
    pl.kernel,
    out_type=jax.ShapeDtypeStruct((NC, N, DEGW), jnp.float32),
    mesh=_mesh,
    scratch_types=[
        [pltpu.VMEM((CH,), jnp.int32)] * 6,
        pltpu.VMEM((CH, DEGW), jnp.float32),
        pltpu.VMEM_SHARED((ACCN, DEGW), jnp.float32),
        [pltpu.SemaphoreType.DMA] * 6,
        [pltpu.SemaphoreType.DMA] * 4,
    ],
    compiler_params=_sc_params,
)
def _deg_kernel(dst_hbm, ones_hbm, out_hbm, dst_v, ones_v, acc, isems, ssems):
    c = lax.axis_index("c")
    s = lax.axis_index("s")
    wid = s * NC + c
    pw = jnp.where(wid < PXT, PW0 + 1, PW0)
    cb = wid * PW0 + jnp.minimum(wid, PXT)
    pltpu.sync_copy(ones_hbm.at[pl.ds(s * RPT, RPT)], acc.at[pl.ds(s * RPT, RPT)])
    pltpu.sync_copy(ones_hbm.at[pl.ds(0, CH)], ones_v)
    plsc.subcore_barrier()

    def idx(ci, q):
        return pltpu.make_async_copy(dst_hbm.at[cb + ci], dst_v[q], isems[q])

    def scat(q, b):
        return pltpu.make_async_copy(ones_v, acc.at[dst_v[q]], ssems[b])

    idx(0, 0).start()
    idx(1, 1).start()

    def body(g, carry):
        for k12 in range(12):
            ci = 12 * g + k12
            q6 = k12 % 6
            k4 = k12 % 4

            @pl.when(ci < pw)
            def _():
                idx(ci, q6).wait()

            @pl.when((ci >= 3) & (ci < pw + 3))
            def _():
                scat((q6 + 3) % 6, (k4 + 1) % 4).wait()

            @pl.when(ci < pw)
            def _():
                scat(q6, k4).start(add=True)

            @pl.when(ci + 2 < pw)
            def _():
                idx(ci + 2, (q6 + 2) % 6).start()

        return carry

    lax.fori_loop(0, (P + 3 + 11) // 12, body, 0)
    plsc.subcore_barrier()
    pltpu.sync_copy(acc.at[pl.ds(s * RPT, RPT)], out_hbm.at[c, pl.ds(s * RPT, RPT)])


@functools.partial(
    pl.kernel,
    out_type=jax.ShapeDtypeStruct((NC, N, D), jnp.float32),
    mesh=_mesh,
    scratch_types=[
        [pltpu.VMEM((CH,), jnp.int32)] * 6,
        [pltpu.VMEM((CH,), jnp.int32)] * 6,
        [pltpu.VMEM((CH, D), jnp.float32)] * 3,
        pltpu.VMEM_SHARED((ACCN, D), jnp.float32),
        [pltpu.SemaphoreType.DMA] * 6,
        [pltpu.SemaphoreType.DMA] * 3,
        [pltpu.SemaphoreType.DMA] * 2,
    ],
    compiler_params=_sc_params,
)
def _conv_kernel(xs_hbm, src_hbm, dst_hbm, out_hbm, src_v, dst_v, rows,
                 acc, isems, gsems, ssems):
    c = lax.axis_index("c")
    s = lax.axis_index("s")
    wid = s * NC + c
    pw = jnp.where(wid < PXT, PW0 + 1, PW0)
    cb = wid * PW0 + jnp.minimum(wid, PXT)

    def idx(ci, q):
        return (pltpu.make_async_copy(src_hbm.at[cb + ci], src_v[q], isems[q]),
                pltpu.make_async_copy(dst_hbm.at[cb + ci], dst_v[q], isems[q]))

    def gath(ci8, b4):
        return pltpu.make_async_copy(xs_hbm.at[src_v[ci8]], rows[b4], gsems[b4])

    def scat(ci8, b4, k):
        return pltpu.make_async_copy(rows[b4], acc.at[dst_v[ci8]], ssems[k])

    for q in range(4):
        for d in idx(q, q):
            d.start()
    for d in idx(0, 0):
        d.wait()
    gath(0, 0).start()
    for d in idx(1, 1):
        d.wait()
    gath(1, 1).start()
    pltpu.sync_copy(xs_hbm.at[pl.ds(s * RPT, RPT)], acc.at[pl.ds(s * RPT, RPT)])
    plsc.subcore_barrier()

    def body(g, carry):
        for k6 in range(6):
            ci = 6 * g + k6
            k3 = k6 % 3
            k = k6 % 2

            @pl.when(ci < pw)
            def _():
                gath(k6, k3).wait()

            @pl.when((ci >= 1) & (ci < pw + 1))
            def _():
                scat((k6 + 5) % 6, (k3 + 2) % 3, 1 - k).wait()

            @pl.when(ci < pw)
            def _():
                scat(k6, k3, k).start(add=True)

            @pl.when(ci + 2 < pw)
            def _():
                for d in idx(ci + 2, (k6 + 2) % 6):
                    d.wait()
                gath((k6 + 2) % 6, (k3 + 2) % 3).start()

            @pl.when(ci + 4 < pw)
            def _():
                for d in idx(ci + 4, (k6 + 4) % 6):
                    d.start()

        return carry

    lax.fori_loop(0, (P + 1 + 5) // 6, body, 0)
    plsc.subcore_barrier()
    pltpu.sync_copy(acc.at[pl.ds(s * RPT, RPT)], out_hbm.at[c, pl.ds(s * RPT, RPT)])


MB = 2000
NBLK = N // MB


def _prep_body(d0_ref, d1_ref, x_ref, dinv_ref, xs_ref):
    deg = d0_ref[...] + d1_ref[...] - 1.0
    dinv = lax.rsqrt(deg)
    dinv_ref[...] = dinv
    xs_ref[...] = x_ref[...] * dinv


_prep = pl.pallas_call(
    _prep_body,
    grid=(NBLK,),
    in_specs=[
        pl.BlockSpec((MB, 1), lambda i: (i, 0)),
        pl.BlockSpec((MB, 1), lambda i: (i, 0)),
        pl.BlockSpec((MB, D), lambda i: (i, 0)),
    ],
    out_specs=(
        pl.BlockSpec((MB, 1), lambda i: (i, 0)),
        pl.BlockSpec((MB, D), lambda i: (i, 0)),
    ),
    out_shape=(
        jax.ShapeDtypeStruct((N, 1), jnp.float32),
        jax.ShapeDtypeStruct((N, D), jnp.float32),
    ),
)


def _mmln_body(residual, scale_out, *refs):
    if residual:
        (p0_ref, p1_ref, xs_ref, dinv_ref, w_ref, b_ref, xres_ref,
         lnw_ref, lnb_ref, out_ref, h_scr, acc_ref) = refs
    else:
        (p0_ref, p1_ref, xs_ref, dinv_ref, w_ref, b_ref,
         lnw_ref, lnb_ref, out_ref, h_scr, acc_ref) = refs
    i = pl.program_id(0)

    @pl.when(i == 0)
    def _():
        acc_ref[0] = 0.0
        acc_ref[1] = 0.0

    @pl.when(i < NBLK)
    def _():
        t = p0_ref[0] + p1_ref[0] - xs_ref[...]
        z = t * dinv_ref[...]
        h = jnp.dot(z, w_ref[...], preferred_element_type=jnp.float32) + b_ref[...]
        if residual:
            h = h + xres_ref[...]
        h_scr[pl.ds(i * MB, MB), :] = h
        acc_ref[0] += jnp.sum(h)
        acc_ref[1] += jnp.sum(h * h)

    @pl.when(i >= NBLK)
    def _():
        inv_n = 1.0 / (N * D)
        mean = acc_ref[0] * inv_n
        var = acc_ref[1] * inv_n - mean * mean
        rstd = lax.rsqrt(var + EPS)
        h = h_scr[pl.ds((i - NBLK) * MB, MB), :]
        y = (h - mean) * rstd * lnw_ref[...] + lnb_ref[...]
        y = jnp.maximum(y, 0.0)
        if scale_out:
            y = y * dinv_ref[...]
        out_ref[...] = y


def _make_mmln(residual, scale_out):
    def ph1_map(i):
        return (jnp.minimum(i, NBLK - 1), 0)

    p0_spec = pl.BlockSpec((1, MB, D), lambda i: (0, jnp.minimum(i, NBLK - 1), 0))
    p1_spec = pl.BlockSpec((1, MB, D), lambda i: (1, jnp.minimum(i, NBLK - 1), 0))
    row1_spec = pl.BlockSpec((MB, D), ph1_map)
    dinv_spec = pl.BlockSpec((MB, 1), lambda i: (i % NBLK, 0))
    full_spec = pl.BlockSpec((D, D), lambda i: (0, 0))
    b_spec = pl.BlockSpec((1, D), lambda i: (0, 0))
    in_specs = [p0_spec, p1_spec, row1_spec, dinv_spec, full_spec, b_spec]
    if residual:
        in_specs.append(row1_spec)
    in_specs += [b_spec, b_spec]
    return pl.pallas_call(
        functools.partial(_mmln_body, residual, scale_out),
        grid=(2 * NBLK,),
        in_specs=in_specs,
        out_specs=pl.BlockSpec(
            (MB, D), lambda i: (jnp.where(i < NBLK, 0, i - NBLK), 0)),
        out_shape=jax.ShapeDtypeStruct((N, D), jnp.float32),
        scratch_shapes=[
            pltpu.VMEM((N, D), jnp.float32),
            pltpu.SMEM((2,), jnp.float32),
        ],
    )


_mmln0 = _make_mmln(False, True)
_mmln1 = _make_mmln(True, False)


def kernel(x, edge_index, W0, b0, W1, b1, ln0_w, ln0_b, ln1_w, ln1_b):
    src = edge_index[0].reshape(NCH, CH)
    dst = edge_index[1].reshape(NCH, CH)
    ones = jnp.ones((N, DEGW), jnp.float32)
    b0r = b0.reshape(1, D)
    b1r = b1.reshape(1, D)
    ln0w = ln0_w.reshape(1, D)
    ln0b = ln0_b.reshape(1, D)
    ln1w = ln1_w.reshape(1, D)
    ln1b = ln1_b.reshape(1, D)

    degp = _deg_kernel(dst, ones)
    dinv, xs0 = _prep(degp[0, :, 0:1], degp[1, :, 0:1], x)

    p = _conv_kernel(xs0, src, dst)
    xs1 = _mmln0(p, p, xs0, dinv, W0, b0r, ln0w, ln0b)

    q = _conv_kernel(xs1, src, dst)
    out = _mmln1(q, q, xs1, dinv, W1, b1r, x, ln1w, ln1b)
    return out
